# baseline (speedup 1.0000x reference)
import functools

import jax
import jax.numpy as jnp
from jax import lax
from jax.experimental import pallas as pl
from jax.experimental.pallas import tpu as pltpu
from jax.experimental.pallas import tpu_sc as plsc

f32 = jnp.float32
i32 = jnp.int32

NP = 10000
NL = 128
D = 128
DO = 64
EC = 320000
EI = 10000

NCORE = 2
NSUB = 16
NW = NCORE * NSUB
NP_PAD = 10240
STRIPE = NP_PAD // NSUB
K = 128
NCH = 80
PW_EC = NCH * K
EC_PAD = PW_EC * NW
NBUF = 2
NGRP = NCH // NBUF
DH = D // 2
NCH2 = EC_PAD // (NSUB * K)
IRING = 4
NGRP4 = NCH2 // IRING
KI = 112
NCH_EI = 3
PW_EI = KI * NCH_EI
EI_PAD = PW_EI * NW
BFLAT = NP_PAD * D
BSTRIPE = BFLAT // NSUB
DUMMY_DST = NP_PAD - 1
DUMMY_ISP = 10200
RB = 1024
NBLK = NP_PAD // RB

_mesh = plsc.VectorSubcoreMesh(core_axis_name="core", subcore_axis_name="subcore")


@functools.partial(
    pl.kernel,
    mesh=_mesh,
    out_type=(
        jax.ShapeDtypeStruct((NCORE, NP_PAD), f32),
        jax.ShapeDtypeStruct((NCORE, BFLAT), f32),
    ),
    scratch_types=[
        pltpu.VMEM_SHARED((NP_PAD,), f32),
        pltpu.VMEM_SHARED((BFLAT,), f32),
        pltpu.VMEM((NCH, K), i32),
        pltpu.VMEM((NCH_EI, KI), i32),
        pltpu.VMEM((NCH_EI, KI), i32),
        pltpu.VMEM((NCH_EI, KI), i32),
        pltpu.VMEM((NCH, K), f32),
        pltpu.VMEM((NCH_EI, KI), f32),
    ],
)
def _sc_prep(dst3_hbm, isrc3_hbm, idst3_hbm, onesa_hbm, onesb_hbm, z_hbm,
             deg_out, b_out, deg_sh, b_sh, idxd2, src2, dstl2, flat2,
             onesa_v, onesb_v):
    c = lax.axis_index("core")
    s = lax.axis_index("subcore")
    w = c * NSUB + s
    pltpu.sync_copy(z_hbm.at[pl.ds(s * BSTRIPE, BSTRIPE)],
                    b_sh.at[pl.ds(s * BSTRIPE, BSTRIPE)])
    pltpu.sync_copy(z_hbm.at[pl.ds(s * STRIPE, STRIPE)],
                    deg_sh.at[pl.ds(s * STRIPE, STRIPE)])
    pltpu.sync_copy(dst3_hbm.at[w], idxd2)
    pltpu.sync_copy(isrc3_hbm.at[w], src2)
    pltpu.sync_copy(idst3_hbm.at[w], dstl2)
    pltpu.sync_copy(onesa_hbm, onesa_v)
    pltpu.sync_copy(onesb_hbm, onesb_v)
    plsc.subcore_barrier()

    @pl.loop(0, NCH)
    def _deg_chunk(g):
        pltpu.sync_copy(onesa_v.at[g], deg_sh.at[idxd2.at[g]], add=True)

    for q in range(NCH_EI):
        for j in range(KI // 16):
            sl = pl.ds(j * 16, 16)
            flat2[q, sl] = src2[q, sl] * D + dstl2[q, sl]
        pltpu.sync_copy(onesb_v.at[q], b_sh.at[flat2.at[q]], add=True)

    plsc.subcore_barrier()
    pltpu.sync_copy(deg_sh.at[pl.ds(s * STRIPE, STRIPE)],
                    deg_out.at[c, pl.ds(s * STRIPE, STRIPE)])
    pltpu.sync_copy(b_sh.at[pl.ds(s * BSTRIPE, BSTRIPE)],
                    b_out.at[c, pl.ds(s * BSTRIPE, BSTRIPE)])


@functools.partial(
    pl.kernel,
    mesh=_mesh,
    out_type=jax.ShapeDtypeStruct((NCORE, NP_PAD, DH), f32),
    compiler_params=pltpu.CompilerParams(use_tc_tiling_on_sc=False),
    scratch_types=[
        pltpu.VMEM_SHARED((NP_PAD, DH), f32),
        pltpu.VMEM_SHARED((NP_PAD, DH), f32),
        pltpu.VMEM((IRING, K), i32),
        pltpu.VMEM((IRING, K), i32),
        pltpu.VMEM((NBUF, K, DH), f32),
        pltpu.SemaphoreType.DMA,
        pltpu.SemaphoreType.DMA,
        pltpu.SemaphoreType.DMA,
    ],
)
def _sc_spmm(x2_hbm, src3_hbm, dst3_hbm, zh_hbm, s_out,
             x_sh, acc_sh, idxs_rv, idxd_rv, rows_v, gsem, isem, ssem):
    c = lax.axis_index("core")
    s = lax.axis_index("subcore")
    r0 = s * STRIPE
    pltpu.sync_copy(x2_hbm.at[c, pl.ds(r0, STRIPE)], x_sh.at[pl.ds(r0, STRIPE)])
    pltpu.sync_copy(zh_hbm.at[pl.ds(r0, STRIPE)], acc_sh.at[pl.ds(r0, STRIPE)])
    plsc.subcore_barrier()

    def _wait_idx(g, t):
        pltpu.make_async_copy(src3_hbm.at[s, g], idxs_rv.at[t], isem).wait()
        pltpu.make_async_copy(dst3_hbm.at[s, g], idxd_rv.at[t], isem).wait()

    def _body(g, j, first, last, refill=True):
        if not last:
            _wait_idx(g + 1, (j + 1) % IRING)
        if not first:
            jp = (j - 1) % IRING
            pltpu.make_async_copy(rows_v.at[(j - 1) % 2],
                                  acc_sh.at[idxd_rv.at[jp]],
                                  ssem).wait()
            if refill:
                pltpu.async_copy(src3_hbm.at[s, g + 3], idxs_rv.at[jp], isem)
                pltpu.async_copy(dst3_hbm.at[s, g + 3], idxd_rv.at[jp], isem)
        if not last:
            pltpu.async_copy(x_sh.at[idxs_rv.at[(j + 1) % IRING]],
                             rows_v.at[(j + 1) % 2], gsem)
        pltpu.make_async_copy(x_sh.at[idxs_rv.at[j]],
                              rows_v.at[j % 2], gsem).wait()
        pltpu.async_copy(rows_v.at[j % 2], acc_sh.at[idxd_rv.at[j]],
                         ssem, add=True)

    for t in range(IRING):
        pltpu.async_copy(src3_hbm.at[s, t], idxs_rv.at[t], isem)
        pltpu.async_copy(dst3_hbm.at[s, t], idxd_rv.at[t], isem)
    _wait_idx(0, 0)
    pltpu.async_copy(x_sh.at[idxs_rv.at[0]], rows_v.at[0], gsem)

    for j in range(IRING):
        _body(j, j, first=(j == 0), last=False)

    @pl.loop(0, NGRP4 - 2)
    def _grp(q):
        g0 = (q + 1) * IRING
        for j in range(IRING):
            _body(g0 + j, j, first=False, last=False)

    g0t = (NGRP4 - 1) * IRING
    for j in range(IRING):
        _body(g0t + j, j, first=False, last=(j == IRING - 1),
              refill=(g0t + j + 3 < NCH2))
    pltpu.make_async_copy(rows_v.at[(IRING - 1) % 2],
                          acc_sh.at[idxd_rv.at[IRING - 1]], ssem).wait()

    plsc.subcore_barrier()
    pltpu.sync_copy(acc_sh.at[pl.ds(r0, STRIPE)],
                    s_out.at[c, pl.ds(r0, STRIPE)])

    plsc.subcore_barrier()
    pltpu.sync_copy(acc_sh.at[pl.ds(r0, STRIPE)],
                    s_out.at[c, pl.ds(r0, STRIPE)])


def _dis_from_deg(degt):
    deg = degt[:, 0:1] + degt[:, 1:2] + 1.0
    return lax.rsqrt(deg)


def _tc_scale_body(degt_ref, xp_ref, xs_ref):
    xs = xp_ref[...] * _dis_from_deg(degt_ref[...])
    xs_ref[0] = xs[:, :DH]
    xs_ref[1] = xs[:, DH:]


_tc_scale = pl.pallas_call(
    _tc_scale_body,
    grid=(NBLK,),
    in_specs=[
        pl.BlockSpec((RB, 2), lambda i: (i, 0)),
        pl.BlockSpec((RB, D), lambda i: (i, 0)),
    ],
    out_specs=pl.BlockSpec((NCORE, RB, DH), lambda i: (0, i, 0)),
    out_shape=jax.ShapeDtypeStruct((NCORE, NP_PAD, DH), f32),
)


def _tc_layer1_body(sp_ref, degt_ref, xp_ref, bp_ref, xl_ref,
                    wg_ref, bg_ref, wrr_ref, brr_ref, wrt_ref,
                    wir_ref, bir_ref, wit_ref,
                    hp_ref, hps_ref, hl_ref, acc_ref):
    i = pl.program_id(0)
    sarr = sp_ref[...]
    sagg = jnp.concatenate([sarr[0], sarr[1]], axis=-1)
    dis = _dis_from_deg(degt_ref[...])
    xp = xp_ref[...]
    barr = bp_ref[...]
    b = barr[0] + barr[1]
    xl = xl_ref[...]

    ax = dis * sagg + (dis * dis) * xp
    gcn = jnp.dot(ax, wg_ref[...], preferred_element_type=f32) + bg_ref[...]
    rev_agg = jnp.dot(b, xl, preferred_element_type=f32)
    rev = (jnp.dot(rev_agg, wrr_ref[...], preferred_element_type=f32)
           + brr_ref[...]
           + jnp.dot(xp, wrt_ref[...], preferred_element_type=f32))
    hp = jnp.maximum(0.5 * (gcn + rev), 0.0)
    rows = lax.broadcasted_iota(i32, (RB, 1), 0) + i * RB
    hp = jnp.where(rows < NP, hp, 0.0)
    hp_ref[...] = hp
    hps = hp * dis
    hps_ref[0] = hps[:, :DH]
    hps_ref[1] = hps[:, DH:]

    @pl.when(i == 0)
    def _():
        acc_ref[...] = jnp.zeros_like(acc_ref)

    acc_ref[...] += lax.dot_general(b, xp, (((0,), (0,)), ((), ())),
                                    preferred_element_type=f32)

    @pl.when(i == NBLK - 1)
    def _():
        hl_ref[...] = jnp.maximum(
            jnp.dot(acc_ref[...], wir_ref[...], preferred_element_type=f32)
            + bir_ref[...]
            + jnp.dot(xl, wit_ref[...], preferred_element_type=f32), 0.0)


_tc_layer1 = pl.pallas_call(
    _tc_layer1_body,
    grid=(NBLK,),
    in_specs=[
        pl.BlockSpec((NCORE, RB, DH), lambda i: (0, i, 0)),
        pl.BlockSpec((RB, 2), lambda i: (i, 0)),
        pl.BlockSpec((RB, D), lambda i: (i, 0)),
        pl.BlockSpec((NCORE, RB, D), lambda i: (0, i, 0)),
        pl.BlockSpec((NL, D), lambda i: (0, 0)),
        pl.BlockSpec((D, D), lambda i: (0, 0)),
        pl.BlockSpec((1, D), lambda i: (0, 0)),
        pl.BlockSpec((D, D), lambda i: (0, 0)),
        pl.BlockSpec((1, D), lambda i: (0, 0)),
        pl.BlockSpec((D, D), lambda i: (0, 0)),
        pl.BlockSpec((D, D), lambda i: (0, 0)),
        pl.BlockSpec((1, D), lambda i: (0, 0)),
        pl.BlockSpec((D, D), lambda i: (0, 0)),
    ],
    out_specs=[
        pl.BlockSpec((RB, D), lambda i: (i, 0)),
        pl.BlockSpec((NCORE, RB, DH), lambda i: (0, i, 0)),
        pl.BlockSpec((NL, D), lambda i: (0, 0)),
    ],
    out_shape=(
        jax.ShapeDtypeStruct((NP_PAD, D), f32),
        jax.ShapeDtypeStruct((NCORE, NP_PAD, DH), f32),
        jax.ShapeDtypeStruct((NL, D), f32),
    ),
    scratch_shapes=[pltpu.VMEM((NL, D), f32)],
)


def _tc_layer2_body(sp_ref, degt_ref, hp_ref, bp_ref, hl_ref,
                    wg_ref, bg_ref, wrr_ref, brr_ref, wrt_ref,
                    wir_ref, bir_ref, wit_ref,
                    op_ref, ol_ref, acc_ref):
    i = pl.program_id(0)
    sarr = sp_ref[...]
    sagg = jnp.concatenate([sarr[0], sarr[1]], axis=-1)
    dis = _dis_from_deg(degt_ref[...])
    hp = hp_ref[...]
    barr = bp_ref[...]
    b = barr[0] + barr[1]
    hl = hl_ref[...]

    ax = dis * sagg + (dis * dis) * hp
    gcn = jnp.dot(ax, wg_ref[...], preferred_element_type=f32) + bg_ref[...]
    rev_agg = jnp.dot(b, hl, preferred_element_type=f32)
    rev = (jnp.dot(rev_agg, wrr_ref[...], preferred_element_type=f32)
           + brr_ref[...]
           + jnp.dot(hp, wrt_ref[...], preferred_element_type=f32))
    op_ref[...] = 0.5 * (gcn + rev)

    @pl.when(i == 0)
    def _():
        acc_ref[...] = jnp.zeros_like(acc_ref)

    acc_ref[...] += lax.dot_general(b, hp, (((0,), (0,)), ((), ())),
                                    preferred_element_type=f32)

    @pl.when(i == NBLK - 1)
    def _():
        ol_ref[...] = (
            jnp.dot(acc_ref[...], wir_ref[...], preferred_element_type=f32)
            + bir_ref[...]
            + jnp.dot(hl, wit_ref[...], preferred_element_type=f32))


_tc_layer2 = pl.pallas_call(
    _tc_layer2_body,
    grid=(NBLK,),
    in_specs=[
        pl.BlockSpec((NCORE, RB, DH), lambda i: (0, i, 0)),
        pl.BlockSpec((RB, 2), lambda i: (i, 0)),
        pl.BlockSpec((RB, D), lambda i: (i, 0)),
        pl.BlockSpec((NCORE, RB, D), lambda i: (0, i, 0)),
        pl.BlockSpec((NL, D), lambda i: (0, 0)),
        pl.BlockSpec((D, DO), lambda i: (0, 0)),
        pl.BlockSpec((1, DO), lambda i: (0, 0)),
        pl.BlockSpec((D, DO), lambda i: (0, 0)),
        pl.BlockSpec((1, DO), lambda i: (0, 0)),
        pl.BlockSpec((D, DO), lambda i: (0, 0)),
        pl.BlockSpec((D, DO), lambda i: (0, 0)),
        pl.BlockSpec((1, DO), lambda i: (0, 0)),
        pl.BlockSpec((D, DO), lambda i: (0, 0)),
    ],
    out_specs=[
        pl.BlockSpec((RB, DO), lambda i: (i, 0)),
        pl.BlockSpec((NL, DO), lambda i: (0, 0)),
    ],
    out_shape=(
        jax.ShapeDtypeStruct((NP_PAD, DO), f32),
        jax.ShapeDtypeStruct((NL, DO), f32),
    ),
    scratch_shapes=[pltpu.VMEM((NL, D), f32)],
)


def kernel(x_paper, x_label, edge_cites, edge_is_src, edge_is_dst,
           l1_gcn_W, l1_gcn_b, l1_is_Wrel, l1_is_brel, l1_is_Wroot,
           l1_rev_Wrel, l1_rev_brel, l1_rev_Wroot,
           l2_gcn_W, l2_gcn_b, l2_is_Wrel, l2_is_brel, l2_is_Wroot,
           l2_rev_Wrel, l2_rev_brel, l2_rev_Wroot):
    src = edge_cites[0]
    dst = edge_cites[1]
    pad_ec = EC_PAD - EC
    src_p = jnp.concatenate([src, jnp.zeros((pad_ec,), i32)]).reshape(NW, NCH, K)
    dst_p = jnp.concatenate([dst, jnp.full((pad_ec,), DUMMY_DST, i32)]
                            ).reshape(NW, NCH, K)
    pad_ei = EI_PAD - EI
    isrc_p = jnp.concatenate([edge_is_src, jnp.full((pad_ei,), DUMMY_ISP, i32)]
                             ).reshape(NW, NCH_EI, KI)
    idst_p = jnp.concatenate([edge_is_dst, jnp.zeros((pad_ei,), i32)]
                             ).reshape(NW, NCH_EI, KI)
    xp_pad = jnp.concatenate([x_paper, jnp.zeros((NP_PAD - NP, D), f32)])
    zeros_flat = jnp.zeros((BFLAT,), f32)
    zeros_h = jnp.zeros((NP_PAD, DH), f32)
    ones_a = jnp.ones((NCH, K), f32)
    ones_b = jnp.ones((NCH_EI, KI), f32)
    src_s = src_p.reshape(NSUB, NCH2, K)
    dst_s = dst_p.reshape(NSUB, NCH2, K)

    deg_part, b_flat = _sc_prep(dst_p, isrc_p, idst_p, ones_a, ones_b, zeros_flat)
    deg_t = deg_part.T
    b_part = b_flat.reshape(NCORE, NP_PAD, D)

    xs2 = _tc_scale(deg_t, xp_pad)
    s1 = _sc_spmm(xs2, src_s, dst_s, zeros_h)
    hp, hps2, hl = _tc_layer1(
        s1, deg_t, xp_pad, b_part, x_label,
        l1_gcn_W, l1_gcn_b.reshape(1, D),
        l1_rev_Wrel, l1_rev_brel.reshape(1, D), l1_rev_Wroot,
        l1_is_Wrel, l1_is_brel.reshape(1, D), l1_is_Wroot)
    s2 = _sc_spmm(hps2, src_s, dst_s, zeros_h)
    op, ol = _tc_layer2(
        s2, deg_t, hp, b_part, hl,
        l2_gcn_W, l2_gcn_b.reshape(1, DO),
        l2_rev_Wrel, l2_rev_brel.reshape(1, DO), l2_rev_Wroot,
        l2_is_Wrel, l2_is_brel.reshape(1, DO), l2_is_Wroot)
    return op[:NP], ol


# speedup vs baseline: 1.0007x; 1.0007x over previous
import functools

import jax
import jax.numpy as jnp
from jax import lax
from jax.experimental import pallas as pl
from jax.experimental.pallas import tpu as pltpu
from jax.experimental.pallas import tpu_sc as plsc

f32 = jnp.float32
i32 = jnp.int32

NP = 10000
NL = 128
D = 128
DO = 64
EC = 320000
EI = 10000

NCORE = 2
NSUB = 16
NW = NCORE * NSUB
NP_PAD = 10240
STRIPE = NP_PAD // NSUB
K = 128
NCH = 80
PW_EC = NCH * K
EC_PAD = PW_EC * NW
NBUF = 2
NGRP = NCH // NBUF
DH = D // 2
NCH2 = EC_PAD // (NSUB * K)
IRING = 4
NGRP4 = NCH2 // IRING
KI = 112
NCH_EI = 3
PW_EI = KI * NCH_EI
EI_PAD = PW_EI * NW
BFLAT = NP_PAD * D
BSTRIPE = BFLAT // NSUB
DUMMY_DST = NP_PAD - 1
DUMMY_ISP = 10200
RB = 1024
NBLK = NP_PAD // RB

_mesh = plsc.VectorSubcoreMesh(core_axis_name="core", subcore_axis_name="subcore")


@functools.partial(
    pl.kernel,
    mesh=_mesh,
    out_type=(
        jax.ShapeDtypeStruct((NCORE, NP_PAD), f32),
        jax.ShapeDtypeStruct((NCORE, BFLAT), f32),
    ),
    scratch_types=[
        pltpu.VMEM_SHARED((NP_PAD,), f32),
        pltpu.VMEM_SHARED((BFLAT,), f32),
        pltpu.VMEM((NCH, K), i32),
        pltpu.VMEM((NCH_EI, KI), i32),
        pltpu.VMEM((NCH_EI, KI), i32),
        pltpu.VMEM((NCH_EI, KI), i32),
        pltpu.VMEM((NCH, K), f32),
        pltpu.VMEM((NCH_EI, KI), f32),
        pltpu.SemaphoreType.DMA,
    ],
)
def _sc_prep(dst3_hbm, isrc3_hbm, idst3_hbm, onesa_hbm, onesb_hbm, z_hbm,
             deg_out, b_out, deg_sh, b_sh, idxd2, src2, dstl2, flat2,
             onesa_v, onesb_v, dsem):
    c = lax.axis_index("core")
    s = lax.axis_index("subcore")
    w = c * NSUB + s
    pltpu.sync_copy(z_hbm.at[pl.ds(s * BSTRIPE, BSTRIPE)],
                    b_sh.at[pl.ds(s * BSTRIPE, BSTRIPE)])
    pltpu.sync_copy(z_hbm.at[pl.ds(s * STRIPE, STRIPE)],
                    deg_sh.at[pl.ds(s * STRIPE, STRIPE)])
    pltpu.sync_copy(dst3_hbm.at[w], idxd2)
    pltpu.sync_copy(isrc3_hbm.at[w], src2)
    pltpu.sync_copy(idst3_hbm.at[w], dstl2)
    pltpu.sync_copy(onesa_hbm, onesa_v)
    pltpu.sync_copy(onesb_hbm, onesb_v)
    plsc.subcore_barrier()

    @pl.loop(0, NCH)
    def _deg_chunk(g):
        pltpu.async_copy(onesa_v.at[g], deg_sh.at[idxd2.at[g]], dsem, add=True)

    for q in range(NCH_EI):
        for j in range(KI // 16):
            sl = pl.ds(j * 16, 16)
            flat2[q, sl] = src2[q, sl] * D + dstl2[q, sl]
        pltpu.async_copy(onesb_v.at[q], b_sh.at[flat2.at[q]], dsem, add=True)

    @pl.loop(0, NCH)
    def _deg_drain(g):
        pltpu.make_async_copy(onesa_v.at[g], deg_sh.at[idxd2.at[g]], dsem).wait()
    for q in range(NCH_EI):
        pltpu.make_async_copy(onesb_v.at[q], b_sh.at[flat2.at[q]], dsem).wait()

    plsc.subcore_barrier()
    pltpu.sync_copy(deg_sh.at[pl.ds(s * STRIPE, STRIPE)],
                    deg_out.at[c, pl.ds(s * STRIPE, STRIPE)])
    pltpu.sync_copy(b_sh.at[pl.ds(s * BSTRIPE, BSTRIPE)],
                    b_out.at[c, pl.ds(s * BSTRIPE, BSTRIPE)])


@functools.partial(
    pl.kernel,
    mesh=_mesh,
    out_type=jax.ShapeDtypeStruct((NCORE, NP_PAD, DH), f32),
    compiler_params=pltpu.CompilerParams(use_tc_tiling_on_sc=False),
    scratch_types=[
        pltpu.VMEM_SHARED((NP_PAD, DH), f32),
        pltpu.VMEM_SHARED((NP_PAD, DH), f32),
        pltpu.VMEM((IRING, K), i32),
        pltpu.VMEM((IRING, K), i32),
        pltpu.VMEM((NBUF, K, DH), f32),
        pltpu.SemaphoreType.DMA,
        pltpu.SemaphoreType.DMA,
    ],
)
def _sc_spmm(x2_hbm, src3_hbm, dst3_hbm, zh_hbm, s_out,
             x_sh, acc_sh, idxs_rv, idxd_rv, rows_v, gsem, isem):
    c = lax.axis_index("core")
    s = lax.axis_index("subcore")
    r0 = s * STRIPE
    pltpu.sync_copy(x2_hbm.at[c, pl.ds(r0, STRIPE)], x_sh.at[pl.ds(r0, STRIPE)])
    pltpu.sync_copy(zh_hbm.at[pl.ds(r0, STRIPE)], acc_sh.at[pl.ds(r0, STRIPE)])
    plsc.subcore_barrier()

    def _wait_idx(g, t):
        pltpu.make_async_copy(src3_hbm.at[s, g], idxs_rv.at[t], isem).wait()
        pltpu.make_async_copy(dst3_hbm.at[s, g], idxd_rv.at[t], isem).wait()

    def _body(g, j, last, refill=True):
        if not last:
            _wait_idx(g + 1, (j + 1) % IRING)
            pltpu.async_copy(x_sh.at[idxs_rv.at[(j + 1) % IRING]],
                             rows_v.at[(j + 1) % 2], gsem)
        pltpu.make_async_copy(x_sh.at[idxs_rv.at[j]],
                              rows_v.at[j % 2], gsem).wait()
        pltpu.sync_copy(rows_v.at[j % 2], acc_sh.at[idxd_rv.at[j]], add=True)
        if refill:
            pltpu.async_copy(src3_hbm.at[s, g + IRING], idxs_rv.at[j], isem)
            pltpu.async_copy(dst3_hbm.at[s, g + IRING], idxd_rv.at[j], isem)

    for t in range(IRING):
        pltpu.async_copy(src3_hbm.at[s, t], idxs_rv.at[t], isem)
        pltpu.async_copy(dst3_hbm.at[s, t], idxd_rv.at[t], isem)
    _wait_idx(0, 0)
    pltpu.async_copy(x_sh.at[idxs_rv.at[0]], rows_v.at[0], gsem)

    @pl.loop(0, NGRP4 - 1)
    def _grp(q):
        g0 = q * IRING
        for j in range(IRING):
            _body(g0 + j, j, last=False)

    g0t = (NGRP4 - 1) * IRING
    for j in range(IRING):
        _body(g0t + j, j, last=(j == IRING - 1), refill=False)

    plsc.subcore_barrier()
    pltpu.sync_copy(acc_sh.at[pl.ds(r0, STRIPE)],
                    s_out.at[c, pl.ds(r0, STRIPE)])

    plsc.subcore_barrier()
    pltpu.sync_copy(acc_sh.at[pl.ds(r0, STRIPE)],
                    s_out.at[c, pl.ds(r0, STRIPE)])


def _dis_from_deg(degt):
    deg = degt[:, 0:1] + degt[:, 1:2] + 1.0
    return lax.rsqrt(deg)


def _tc_scale_body(degt_ref, xp_ref, xs_ref):
    xs = xp_ref[...] * _dis_from_deg(degt_ref[...])
    xs_ref[0] = xs[:, :DH]
    xs_ref[1] = xs[:, DH:]


_tc_scale = pl.pallas_call(
    _tc_scale_body,
    grid=(NBLK,),
    in_specs=[
        pl.BlockSpec((RB, 2), lambda i: (i, 0)),
        pl.BlockSpec((RB, D), lambda i: (i, 0)),
    ],
    out_specs=pl.BlockSpec((NCORE, RB, DH), lambda i: (0, i, 0)),
    out_shape=jax.ShapeDtypeStruct((NCORE, NP_PAD, DH), f32),
)


def _tc_layer1_body(sp_ref, degt_ref, xp_ref, bp_ref, xl_ref,
                    wg_ref, bg_ref, wrr_ref, brr_ref, wrt_ref,
                    wir_ref, bir_ref, wit_ref,
                    hp_ref, hps_ref, hl_ref, acc_ref):
    i = pl.program_id(0)
    sarr = sp_ref[...]
    sagg = jnp.concatenate([sarr[0], sarr[1]], axis=-1)
    dis = _dis_from_deg(degt_ref[...])
    xp = xp_ref[...]
    barr = bp_ref[...]
    b = barr[0] + barr[1]
    xl = xl_ref[...]

    ax = dis * sagg + (dis * dis) * xp
    gcn = jnp.dot(ax, wg_ref[...], preferred_element_type=f32) + bg_ref[...]
    rev_agg = jnp.dot(b, xl, preferred_element_type=f32)
    rev = (jnp.dot(rev_agg, wrr_ref[...], preferred_element_type=f32)
           + brr_ref[...]
           + jnp.dot(xp, wrt_ref[...], preferred_element_type=f32))
    hp = jnp.maximum(0.5 * (gcn + rev), 0.0)
    rows = lax.broadcasted_iota(i32, (RB, 1), 0) + i * RB
    hp = jnp.where(rows < NP, hp, 0.0)
    hp_ref[...] = hp
    hps = hp * dis
    hps_ref[0] = hps[:, :DH]
    hps_ref[1] = hps[:, DH:]

    @pl.when(i == 0)
    def _():
        acc_ref[...] = jnp.zeros_like(acc_ref)

    acc_ref[...] += lax.dot_general(b, xp, (((0,), (0,)), ((), ())),
                                    preferred_element_type=f32)

    @pl.when(i == NBLK - 1)
    def _():
        hl_ref[...] = jnp.maximum(
            jnp.dot(acc_ref[...], wir_ref[...], preferred_element_type=f32)
            + bir_ref[...]
            + jnp.dot(xl, wit_ref[...], preferred_element_type=f32), 0.0)


_tc_layer1 = pl.pallas_call(
    _tc_layer1_body,
    grid=(NBLK,),
    in_specs=[
        pl.BlockSpec((NCORE, RB, DH), lambda i: (0, i, 0)),
        pl.BlockSpec((RB, 2), lambda i: (i, 0)),
        pl.BlockSpec((RB, D), lambda i: (i, 0)),
        pl.BlockSpec((NCORE, RB, D), lambda i: (0, i, 0)),
        pl.BlockSpec((NL, D), lambda i: (0, 0)),
        pl.BlockSpec((D, D), lambda i: (0, 0)),
        pl.BlockSpec((1, D), lambda i: (0, 0)),
        pl.BlockSpec((D, D), lambda i: (0, 0)),
        pl.BlockSpec((1, D), lambda i: (0, 0)),
        pl.BlockSpec((D, D), lambda i: (0, 0)),
        pl.BlockSpec((D, D), lambda i: (0, 0)),
        pl.BlockSpec((1, D), lambda i: (0, 0)),
        pl.BlockSpec((D, D), lambda i: (0, 0)),
    ],
    out_specs=[
        pl.BlockSpec((RB, D), lambda i: (i, 0)),
        pl.BlockSpec((NCORE, RB, DH), lambda i: (0, i, 0)),
        pl.BlockSpec((NL, D), lambda i: (0, 0)),
    ],
    out_shape=(
        jax.ShapeDtypeStruct((NP_PAD, D), f32),
        jax.ShapeDtypeStruct((NCORE, NP_PAD, DH), f32),
        jax.ShapeDtypeStruct((NL, D), f32),
    ),
    scratch_shapes=[pltpu.VMEM((NL, D), f32)],
)


def _tc_layer2_body(sp_ref, degt_ref, hp_ref, bp_ref, hl_ref,
                    wg_ref, bg_ref, wrr_ref, brr_ref, wrt_ref,
                    wir_ref, bir_ref, wit_ref,
                    op_ref, ol_ref, acc_ref):
    i = pl.program_id(0)
    sarr = sp_ref[...]
    sagg = jnp.concatenate([sarr[0], sarr[1]], axis=-1)
    dis = _dis_from_deg(degt_ref[...])
    hp = hp_ref[...]
    barr = bp_ref[...]
    b = barr[0] + barr[1]
    hl = hl_ref[...]

    ax = dis * sagg + (dis * dis) * hp
    gcn = jnp.dot(ax, wg_ref[...], preferred_element_type=f32) + bg_ref[...]
    rev_agg = jnp.dot(b, hl, preferred_element_type=f32)
    rev = (jnp.dot(rev_agg, wrr_ref[...], preferred_element_type=f32)
           + brr_ref[...]
           + jnp.dot(hp, wrt_ref[...], preferred_element_type=f32))
    op_ref[...] = 0.5 * (gcn + rev)

    @pl.when(i == 0)
    def _():
        acc_ref[...] = jnp.zeros_like(acc_ref)

    acc_ref[...] += lax.dot_general(b, hp, (((0,), (0,)), ((), ())),
                                    preferred_element_type=f32)

    @pl.when(i == NBLK - 1)
    def _():
        ol_ref[...] = (
            jnp.dot(acc_ref[...], wir_ref[...], preferred_element_type=f32)
            + bir_ref[...]
            + jnp.dot(hl, wit_ref[...], preferred_element_type=f32))


_tc_layer2 = pl.pallas_call(
    _tc_layer2_body,
    grid=(NBLK,),
    in_specs=[
        pl.BlockSpec((NCORE, RB, DH), lambda i: (0, i, 0)),
        pl.BlockSpec((RB, 2), lambda i: (i, 0)),
        pl.BlockSpec((RB, D), lambda i: (i, 0)),
        pl.BlockSpec((NCORE, RB, D), lambda i: (0, i, 0)),
        pl.BlockSpec((NL, D), lambda i: (0, 0)),
        pl.BlockSpec((D, DO), lambda i: (0, 0)),
        pl.BlockSpec((1, DO), lambda i: (0, 0)),
        pl.BlockSpec((D, DO), lambda i: (0, 0)),
        pl.BlockSpec((1, DO), lambda i: (0, 0)),
        pl.BlockSpec((D, DO), lambda i: (0, 0)),
        pl.BlockSpec((D, DO), lambda i: (0, 0)),
        pl.BlockSpec((1, DO), lambda i: (0, 0)),
        pl.BlockSpec((D, DO), lambda i: (0, 0)),
    ],
    out_specs=[
        pl.BlockSpec((RB, DO), lambda i: (i, 0)),
        pl.BlockSpec((NL, DO), lambda i: (0, 0)),
    ],
    out_shape=(
        jax.ShapeDtypeStruct((NP_PAD, DO), f32),
        jax.ShapeDtypeStruct((NL, DO), f32),
    ),
    scratch_shapes=[pltpu.VMEM((NL, D), f32)],
)


def kernel(x_paper, x_label, edge_cites, edge_is_src, edge_is_dst,
           l1_gcn_W, l1_gcn_b, l1_is_Wrel, l1_is_brel, l1_is_Wroot,
           l1_rev_Wrel, l1_rev_brel, l1_rev_Wroot,
           l2_gcn_W, l2_gcn_b, l2_is_Wrel, l2_is_brel, l2_is_Wroot,
           l2_rev_Wrel, l2_rev_brel, l2_rev_Wroot):
    src = edge_cites[0]
    dst = edge_cites[1]
    pad_ec = EC_PAD - EC
    src_p = jnp.concatenate([src, jnp.zeros((pad_ec,), i32)]).reshape(NW, NCH, K)
    dst_p = jnp.concatenate([dst, jnp.full((pad_ec,), DUMMY_DST, i32)]
                            ).reshape(NW, NCH, K)
    pad_ei = EI_PAD - EI
    isrc_p = jnp.concatenate([edge_is_src, jnp.full((pad_ei,), DUMMY_ISP, i32)]
                             ).reshape(NW, NCH_EI, KI)
    idst_p = jnp.concatenate([edge_is_dst, jnp.zeros((pad_ei,), i32)]
                             ).reshape(NW, NCH_EI, KI)
    xp_pad = jnp.concatenate([x_paper, jnp.zeros((NP_PAD - NP, D), f32)])
    zeros_flat = jnp.zeros((BFLAT,), f32)
    zeros_h = jnp.zeros((NP_PAD, DH), f32)
    ones_a = jnp.ones((NCH, K), f32)
    ones_b = jnp.ones((NCH_EI, KI), f32)
    src_s = src_p.reshape(NSUB, NCH2, K)
    dst_s = dst_p.reshape(NSUB, NCH2, K)

    deg_part, b_flat = _sc_prep(dst_p, isrc_p, idst_p, ones_a, ones_b, zeros_flat)
    deg_t = deg_part.T
    b_part = b_flat.reshape(NCORE, NP_PAD, D)

    xs2 = _tc_scale(deg_t, xp_pad)
    s1 = _sc_spmm(xs2, src_s, dst_s, zeros_h)
    hp, hps2, hl = _tc_layer1(
        s1, deg_t, xp_pad, b_part, x_label,
        l1_gcn_W, l1_gcn_b.reshape(1, D),
        l1_rev_Wrel, l1_rev_brel.reshape(1, D), l1_rev_Wroot,
        l1_is_Wrel, l1_is_brel.reshape(1, D), l1_is_Wroot)
    s2 = _sc_spmm(hps2, src_s, dst_s, zeros_h)
    op, ol = _tc_layer2(
        s2, deg_t, hp, b_part, hl,
        l2_gcn_W, l2_gcn_b.reshape(1, DO),
        l2_rev_Wrel, l2_rev_brel.reshape(1, DO), l2_rev_Wroot,
        l2_is_Wrel, l2_is_brel.reshape(1, DO), l2_is_Wroot)
    return op[:NP], ol


# speedup vs baseline: 1.1143x; 1.1135x over previous
import functools

import jax
import jax.numpy as jnp
from jax import lax
from jax.experimental import pallas as pl
from jax.experimental.pallas import tpu as pltpu
from jax.experimental.pallas import tpu_sc as plsc

f32 = jnp.float32
i32 = jnp.int32

NP = 10000
NL = 128
D = 128
DO = 64
EC = 320000
EI = 10000

NCORE = 2
NSUB = 16
NW = NCORE * NSUB
NP_PAD = 10240
STRIPE = NP_PAD // NSUB
K = 128
NCH = 80
PW_EC = NCH * K
EC_PAD = PW_EC * NW
NBUF = 2
NGRP = NCH // NBUF
DH = D // 2
NCH2 = EC_PAD // (NSUB * K)
IRING = 4
NGRP4 = NCH2 // IRING
KI = 112
NCH_EI = 3
PW_EI = KI * NCH_EI
EI_PAD = PW_EI * NW
BFLAT = NP_PAD * D
BSTRIPE = BFLAT // NSUB
DUMMY_DST = NP_PAD - 1
DUMMY_ISP = 10200
RB = 1024
NBLK = NP_PAD // RB

_mesh = plsc.VectorSubcoreMesh(core_axis_name="core", subcore_axis_name="subcore")


@functools.partial(
    pl.kernel,
    mesh=_mesh,
    out_type=(
        jax.ShapeDtypeStruct((NCORE, NP_PAD), f32),
        jax.ShapeDtypeStruct((NCORE, BFLAT), f32),
    ),
    scratch_types=[
        pltpu.VMEM_SHARED((NP_PAD,), f32),
        pltpu.VMEM_SHARED((BFLAT,), f32),
        pltpu.VMEM((NCH, K), i32),
        pltpu.VMEM((NCH_EI, KI), i32),
        pltpu.VMEM((NCH_EI, KI), i32),
        pltpu.VMEM((NCH_EI, KI), i32),
        pltpu.VMEM((NCH, K), f32),
        pltpu.VMEM((NCH_EI, KI), f32),
        pltpu.SemaphoreType.DMA,
    ],
)
def _sc_prep(dst3_hbm, isrc3_hbm, idst3_hbm, onesa_hbm, onesb_hbm, z_hbm,
             deg_out, b_out, deg_sh, b_sh, idxd2, src2, dstl2, flat2,
             onesa_v, onesb_v, dsem):
    c = lax.axis_index("core")
    s = lax.axis_index("subcore")
    w = c * NSUB + s
    pltpu.sync_copy(z_hbm.at[pl.ds(s * BSTRIPE, BSTRIPE)],
                    b_sh.at[pl.ds(s * BSTRIPE, BSTRIPE)])
    pltpu.sync_copy(z_hbm.at[pl.ds(s * STRIPE, STRIPE)],
                    deg_sh.at[pl.ds(s * STRIPE, STRIPE)])
    pltpu.sync_copy(dst3_hbm.at[w], idxd2)
    pltpu.sync_copy(isrc3_hbm.at[w], src2)
    pltpu.sync_copy(idst3_hbm.at[w], dstl2)
    pltpu.sync_copy(onesa_hbm, onesa_v)
    pltpu.sync_copy(onesb_hbm, onesb_v)
    plsc.subcore_barrier()

    @pl.loop(0, NCH)
    def _deg_chunk(g):
        pltpu.async_copy(onesa_v.at[g], deg_sh.at[idxd2.at[g]], dsem, add=True)

    for q in range(NCH_EI):
        for j in range(KI // 16):
            sl = pl.ds(j * 16, 16)
            flat2[q, sl] = src2[q, sl] * D + dstl2[q, sl]
        pltpu.async_copy(onesb_v.at[q], b_sh.at[flat2.at[q]], dsem, add=True)

    @pl.loop(0, NCH)
    def _deg_drain(g):
        pltpu.make_async_copy(onesa_v.at[g], deg_sh.at[idxd2.at[g]], dsem).wait()
    for q in range(NCH_EI):
        pltpu.make_async_copy(onesb_v.at[q], b_sh.at[flat2.at[q]], dsem).wait()

    plsc.subcore_barrier()
    pltpu.sync_copy(deg_sh.at[pl.ds(s * STRIPE, STRIPE)],
                    deg_out.at[c, pl.ds(s * STRIPE, STRIPE)])
    pltpu.sync_copy(b_sh.at[pl.ds(s * BSTRIPE, BSTRIPE)],
                    b_out.at[c, pl.ds(s * BSTRIPE, BSTRIPE)])


@functools.partial(
    pl.kernel,
    mesh=_mesh,
    out_type=jax.ShapeDtypeStruct((NP_PAD, D), f32),
    compiler_params=pltpu.CompilerParams(use_tc_tiling_on_sc=False),
    scratch_types=[
        pltpu.VMEM_SHARED((NP_PAD, DH), f32),
        pltpu.VMEM_SHARED((NP_PAD, DH), f32),
        pltpu.VMEM((IRING, K), i32),
        pltpu.VMEM((IRING, K), i32),
        pltpu.VMEM((NBUF, K, DH), f32),
        pltpu.SemaphoreType.DMA,
        pltpu.SemaphoreType.DMA,
    ],
)
def _sc_spmm(x2_hbm, src3_hbm, dst3_hbm, zh_hbm, s_out,
             x_sh, acc_sh, idxs_rv, idxd_rv, rows_v, gsem, isem):
    c = lax.axis_index("core")
    s = lax.axis_index("subcore")
    r0 = s * STRIPE
    col0 = pl.multiple_of(c * DH, DH)
    pltpu.sync_copy(x2_hbm.at[pl.ds(r0, STRIPE), pl.ds(col0, DH)],
                    x_sh.at[pl.ds(r0, STRIPE)])
    pltpu.sync_copy(zh_hbm.at[pl.ds(r0, STRIPE)], acc_sh.at[pl.ds(r0, STRIPE)])
    plsc.subcore_barrier()

    def _wait_idx(g, t):
        pltpu.make_async_copy(src3_hbm.at[s, g], idxs_rv.at[t], isem).wait()
        pltpu.make_async_copy(dst3_hbm.at[s, g], idxd_rv.at[t], isem).wait()

    def _body(g, j, last, refill=True):
        if not last:
            _wait_idx(g + 1, (j + 1) % IRING)
            pltpu.async_copy(x_sh.at[idxs_rv.at[(j + 1) % IRING]],
                             rows_v.at[(j + 1) % 2], gsem)
        pltpu.make_async_copy(x_sh.at[idxs_rv.at[j]],
                              rows_v.at[j % 2], gsem).wait()
        pltpu.sync_copy(rows_v.at[j % 2], acc_sh.at[idxd_rv.at[j]], add=True)
        if refill:
            pltpu.async_copy(src3_hbm.at[s, g + IRING], idxs_rv.at[j], isem)
            pltpu.async_copy(dst3_hbm.at[s, g + IRING], idxd_rv.at[j], isem)

    for t in range(IRING):
        pltpu.async_copy(src3_hbm.at[s, t], idxs_rv.at[t], isem)
        pltpu.async_copy(dst3_hbm.at[s, t], idxd_rv.at[t], isem)
    _wait_idx(0, 0)
    pltpu.async_copy(x_sh.at[idxs_rv.at[0]], rows_v.at[0], gsem)

    @pl.loop(0, NGRP4 - 1)
    def _grp(q):
        g0 = q * IRING
        for j in range(IRING):
            _body(g0 + j, j, last=False)

    g0t = (NGRP4 - 1) * IRING
    for j in range(IRING):
        _body(g0t + j, j, last=(j == IRING - 1), refill=False)

    plsc.subcore_barrier()
    pltpu.sync_copy(acc_sh.at[pl.ds(r0, STRIPE)],
                    s_out.at[pl.ds(r0, STRIPE), pl.ds(col0, DH)])


def _dis_from_deg(degt):
    deg = degt[:, 0:1] + degt[:, 1:2] + 1.0
    return lax.rsqrt(deg)


def _tc_scale_body(degt_ref, xp_ref, xs_ref):
    xs_ref[...] = xp_ref[...] * _dis_from_deg(degt_ref[...])


_tc_scale = pl.pallas_call(
    _tc_scale_body,
    grid=(NBLK,),
    in_specs=[
        pl.BlockSpec((RB, 2), lambda i: (i, 0)),
        pl.BlockSpec((RB, D), lambda i: (i, 0)),
    ],
    out_specs=pl.BlockSpec((RB, D), lambda i: (i, 0)),
    out_shape=jax.ShapeDtypeStruct((NP_PAD, D), f32),
)


def _tc_layer1_body(sp_ref, degt_ref, xp_ref, bp_ref, xl_ref,
                    wg_ref, bg_ref, wrr_ref, brr_ref, wrt_ref,
                    wir_ref, bir_ref, wit_ref,
                    hp_ref, hps_ref, hl_ref, acc_ref):
    i = pl.program_id(0)
    sagg = sp_ref[...]
    dis = _dis_from_deg(degt_ref[...])
    xp = xp_ref[...]
    barr = bp_ref[...]
    b = barr[0] + barr[1]
    xl = xl_ref[...]

    ax = dis * sagg + (dis * dis) * xp
    gcn = jnp.dot(ax, wg_ref[...], preferred_element_type=f32) + bg_ref[...]
    rev_agg = jnp.dot(b, xl, preferred_element_type=f32)
    rev = (jnp.dot(rev_agg, wrr_ref[...], preferred_element_type=f32)
           + brr_ref[...]
           + jnp.dot(xp, wrt_ref[...], preferred_element_type=f32))
    hp = jnp.maximum(0.5 * (gcn + rev), 0.0)
    rows = lax.broadcasted_iota(i32, (RB, 1), 0) + i * RB
    hp = jnp.where(rows < NP, hp, 0.0)
    hp_ref[...] = hp
    hps_ref[...] = hp * dis

    @pl.when(i == 0)
    def _():
        acc_ref[...] = jnp.zeros_like(acc_ref)

    acc_ref[...] += lax.dot_general(b, xp, (((0,), (0,)), ((), ())),
                                    preferred_element_type=f32)

    @pl.when(i == NBLK - 1)
    def _():
        hl_ref[...] = jnp.maximum(
            jnp.dot(acc_ref[...], wir_ref[...], preferred_element_type=f32)
            + bir_ref[...]
            + jnp.dot(xl, wit_ref[...], preferred_element_type=f32), 0.0)


_tc_layer1 = pl.pallas_call(
    _tc_layer1_body,
    grid=(NBLK,),
    in_specs=[
        pl.BlockSpec((RB, D), lambda i: (i, 0)),
        pl.BlockSpec((RB, 2), lambda i: (i, 0)),
        pl.BlockSpec((RB, D), lambda i: (i, 0)),
        pl.BlockSpec((NCORE, RB, D), lambda i: (0, i, 0)),
        pl.BlockSpec((NL, D), lambda i: (0, 0)),
        pl.BlockSpec((D, D), lambda i: (0, 0)),
        pl.BlockSpec((1, D), lambda i: (0, 0)),
        pl.BlockSpec((D, D), lambda i: (0, 0)),
        pl.BlockSpec((1, D), lambda i: (0, 0)),
        pl.BlockSpec((D, D), lambda i: (0, 0)),
        pl.BlockSpec((D, D), lambda i: (0, 0)),
        pl.BlockSpec((1, D), lambda i: (0, 0)),
        pl.BlockSpec((D, D), lambda i: (0, 0)),
    ],
    out_specs=[
        pl.BlockSpec((RB, D), lambda i: (i, 0)),
        pl.BlockSpec((RB, D), lambda i: (i, 0)),
        pl.BlockSpec((NL, D), lambda i: (0, 0)),
    ],
    out_shape=(
        jax.ShapeDtypeStruct((NP_PAD, D), f32),
        jax.ShapeDtypeStruct((NP_PAD, D), f32),
        jax.ShapeDtypeStruct((NL, D), f32),
    ),
    scratch_shapes=[pltpu.VMEM((NL, D), f32)],
)


def _tc_layer2_body(sp_ref, degt_ref, hp_ref, bp_ref, hl_ref,
                    wg_ref, bg_ref, wrr_ref, brr_ref, wrt_ref,
                    wir_ref, bir_ref, wit_ref,
                    op_ref, ol_ref, acc_ref):
    i = pl.program_id(0)
    sagg = sp_ref[...]
    dis = _dis_from_deg(degt_ref[...])
    hp = hp_ref[...]
    barr = bp_ref[...]
    b = barr[0] + barr[1]
    hl = hl_ref[...]

    ax = dis * sagg + (dis * dis) * hp
    gcn = jnp.dot(ax, wg_ref[...], preferred_element_type=f32) + bg_ref[...]
    rev_agg = jnp.dot(b, hl, preferred_element_type=f32)
    rev = (jnp.dot(rev_agg, wrr_ref[...], preferred_element_type=f32)
           + brr_ref[...]
           + jnp.dot(hp, wrt_ref[...], preferred_element_type=f32))
    op_ref[...] = 0.5 * (gcn + rev)

    @pl.when(i == 0)
    def _():
        acc_ref[...] = jnp.zeros_like(acc_ref)

    acc_ref[...] += lax.dot_general(b, hp, (((0,), (0,)), ((), ())),
                                    preferred_element_type=f32)

    @pl.when(i == NBLK - 1)
    def _():
        ol_ref[...] = (
            jnp.dot(acc_ref[...], wir_ref[...], preferred_element_type=f32)
            + bir_ref[...]
            + jnp.dot(hl, wit_ref[...], preferred_element_type=f32))


_tc_layer2 = pl.pallas_call(
    _tc_layer2_body,
    grid=(NBLK,),
    in_specs=[
        pl.BlockSpec((RB, D), lambda i: (i, 0)),
        pl.BlockSpec((RB, 2), lambda i: (i, 0)),
        pl.BlockSpec((RB, D), lambda i: (i, 0)),
        pl.BlockSpec((NCORE, RB, D), lambda i: (0, i, 0)),
        pl.BlockSpec((NL, D), lambda i: (0, 0)),
        pl.BlockSpec((D, DO), lambda i: (0, 0)),
        pl.BlockSpec((1, DO), lambda i: (0, 0)),
        pl.BlockSpec((D, DO), lambda i: (0, 0)),
        pl.BlockSpec((1, DO), lambda i: (0, 0)),
        pl.BlockSpec((D, DO), lambda i: (0, 0)),
        pl.BlockSpec((D, DO), lambda i: (0, 0)),
        pl.BlockSpec((1, DO), lambda i: (0, 0)),
        pl.BlockSpec((D, DO), lambda i: (0, 0)),
    ],
    out_specs=[
        pl.BlockSpec((RB, DO), lambda i: (i, 0)),
        pl.BlockSpec((NL, DO), lambda i: (0, 0)),
    ],
    out_shape=(
        jax.ShapeDtypeStruct((NP_PAD, DO), f32),
        jax.ShapeDtypeStruct((NL, DO), f32),
    ),
    scratch_shapes=[pltpu.VMEM((NL, D), f32)],
)


def kernel(x_paper, x_label, edge_cites, edge_is_src, edge_is_dst,
           l1_gcn_W, l1_gcn_b, l1_is_Wrel, l1_is_brel, l1_is_Wroot,
           l1_rev_Wrel, l1_rev_brel, l1_rev_Wroot,
           l2_gcn_W, l2_gcn_b, l2_is_Wrel, l2_is_brel, l2_is_Wroot,
           l2_rev_Wrel, l2_rev_brel, l2_rev_Wroot):
    src = edge_cites[0]
    dst = edge_cites[1]
    pad_ec = EC_PAD - EC
    src_p = jnp.concatenate([src, jnp.zeros((pad_ec,), i32)]).reshape(NW, NCH, K)
    dst_p = jnp.concatenate([dst, jnp.full((pad_ec,), DUMMY_DST, i32)]
                            ).reshape(NW, NCH, K)
    pad_ei = EI_PAD - EI
    isrc_p = jnp.concatenate([edge_is_src, jnp.full((pad_ei,), DUMMY_ISP, i32)]
                             ).reshape(NW, NCH_EI, KI)
    idst_p = jnp.concatenate([edge_is_dst, jnp.zeros((pad_ei,), i32)]
                             ).reshape(NW, NCH_EI, KI)
    xp_pad = jnp.concatenate([x_paper, jnp.zeros((NP_PAD - NP, D), f32)])
    zeros_flat = jnp.zeros((BFLAT,), f32)
    zeros_h = jnp.zeros((NP_PAD, DH), f32)
    ones_a = jnp.ones((NCH, K), f32)
    ones_b = jnp.ones((NCH_EI, KI), f32)
    src_s = src_p.reshape(NSUB, NCH2, K)
    dst_s = dst_p.reshape(NSUB, NCH2, K)

    deg_part, b_flat = _sc_prep(dst_p, isrc_p, idst_p, ones_a, ones_b, zeros_flat)
    deg_t = deg_part.T
    b_part = b_flat.reshape(NCORE, NP_PAD, D)

    xs = _tc_scale(deg_t, xp_pad)
    s1 = _sc_spmm(xs, src_s, dst_s, zeros_h)
    hp, hps, hl = _tc_layer1(
        s1, deg_t, xp_pad, b_part, x_label,
        l1_gcn_W, l1_gcn_b.reshape(1, D),
        l1_rev_Wrel, l1_rev_brel.reshape(1, D), l1_rev_Wroot,
        l1_is_Wrel, l1_is_brel.reshape(1, D), l1_is_Wroot)
    s2 = _sc_spmm(hps, src_s, dst_s, zeros_h)
    op, ol = _tc_layer2(
        s2, deg_t, hp, b_part, hl,
        l2_gcn_W, l2_gcn_b.reshape(1, DO),
        l2_rev_Wrel, l2_rev_brel.reshape(1, DO), l2_rev_Wroot,
        l2_is_Wrel, l2_is_brel.reshape(1, DO), l2_is_Wroot)
    return op[:NP], ol
